# baseline (speedup 1.0000x reference)
import functools

import jax
import jax.numpy as jnp
from jax.experimental import pallas as pl
from jax.experimental.pallas import tpu as pltpu

_B = 1024
_K = 400.0
_N = 500.0
_EPS = 0.1
_ITERS = 20

_NT = (((1,), (1,)), ((), ()))


def _soft_topk_mul(s):
    sm1 = s - 1.0
    c = jnp.maximum(s * s, sm1 * sm1)
    a = 1.0 / (_EPS * jnp.max(c))
    q = jnp.exp((2.0 * s - 1.0) * a)

    ones_col = jnp.ones((q.shape[1], 8), jnp.float32)

    def body(_, winv):
        t = 1.0 / (q + winv)
        ss = jax.lax.dot_general(t, ones_col, (((1,), (0,)), ((), ())),
                                 preferred_element_type=jnp.float32)[:, 0:1]
        return (_N - _K) * (_N - winv * ss) / (_K * ss)

    winv = jax.lax.fori_loop(0, _ITERS, body,
                             jnp.ones((_B, 1), jnp.float32))
    mask = 1.0 - winv / (q + winv)
    return s * mask


def _dot_nt(a, b):
    return jax.lax.dot_general(a, b, _NT, preferred_element_type=jnp.float32)


def _fwd(x_ref, w1_ref, b1_ref, w2_ref, b2_ref, w3_ref, b3_ref, w4_ref,
         b4_ref, o_ref):
    s = jnp.maximum(_dot_nt(x_ref[...], w1_ref[...]) + b1_ref[...], 0.0)
    for w_ref, b_ref in ((w2_ref, b2_ref), (w3_ref, b3_ref)):
        h = _soft_topk_mul(s)
        s = jnp.maximum(_dot_nt(h, w_ref[...]) + b_ref[...], 0.0)
    h = _soft_topk_mul(s)
    o_ref[...] = _dot_nt(h, w4_ref[...]) + b4_ref[...]


@jax.jit
def kernel(x, W1, b1, W2, b2, W3, b3, W4, b4):
    return pl.pallas_call(
        _fwd,
        out_shape=jax.ShapeDtypeStruct((_B, W4.shape[0]), jnp.float32),
    )(x, W1, b1.reshape(1, -1), W2, b2.reshape(1, -1), W3, b3.reshape(1, -1),
      W4, b4.reshape(1, -1))


# speedup vs baseline: 1.6697x; 1.6697x over previous
import functools

import jax
import jax.numpy as jnp
from jax.experimental import pallas as pl
from jax.experimental.pallas import tpu as pltpu

_B = 1024
_K = 400.0
_N = 500.0
_EPS = 0.1
_ITERS = 8

_NT = (((1,), (1,)), ((), ()))


def _soft_topk_mul(s):
    sm1 = s - 1.0
    c = jnp.maximum(s * s, sm1 * sm1)
    a = 1.0 / (_EPS * jnp.max(c))
    q = jnp.exp((2.0 * s - 1.0) * a)

    def body(_, x):
        t = 1.0 / (q + x)
        s1 = jnp.sum(t, axis=1, keepdims=True)
        fprime = jnp.sum((1.0 - x * t) * t, axis=1, keepdims=True)
        xn = x - (x * s1 - (_N - _K)) / fprime
        return jnp.clip(xn, 1e-8, 1e9)

    x = jax.lax.fori_loop(0, _ITERS, body,
                          jnp.full((_B, 1), 1e-6, jnp.float32))
    mask = 1.0 - x / (q + x)
    return s * mask


def _dot_nt(a, b):
    return jax.lax.dot_general(a, b, _NT, preferred_element_type=jnp.float32)


def _fwd(x_ref, w1_ref, b1_ref, w2_ref, b2_ref, w3_ref, b3_ref, w4_ref,
         b4_ref, o_ref):
    s = jnp.maximum(_dot_nt(x_ref[...], w1_ref[...]) + b1_ref[...], 0.0)
    for w_ref, b_ref in ((w2_ref, b2_ref), (w3_ref, b3_ref)):
        h = _soft_topk_mul(s)
        s = jnp.maximum(_dot_nt(h, w_ref[...]) + b_ref[...], 0.0)
    h = _soft_topk_mul(s)
    o_ref[...] = _dot_nt(h, w4_ref[...]) + b4_ref[...]


@jax.jit
def kernel(x, W1, b1, W2, b2, W3, b3, W4, b4):
    return pl.pallas_call(
        _fwd,
        out_shape=jax.ShapeDtypeStruct((_B, W4.shape[0]), jnp.float32),
    )(x, W1, b1.reshape(1, -1), W2, b2.reshape(1, -1), W3, b3.reshape(1, -1),
      W4, b4.reshape(1, -1))


# speedup vs baseline: 1.8483x; 1.1070x over previous
import functools

import jax
import jax.numpy as jnp
from jax.experimental import pallas as pl
from jax.experimental.pallas import tpu as pltpu

_B = 1024
_K = 400.0
_N = 500.0
_EPS = 0.1
_ITERS = 8

_NT = (((1,), (1,)), ((), ()))


def _soft_topk_mul(s):
    m = jnp.max(jnp.maximum(s, jnp.abs(s - 1.0)))
    a = 1.0 / (_EPS * m * m)
    q = jnp.exp((2.0 * s - 1.0) * a)

    def body(_, x):
        t = 1.0 / (q + x)
        s1 = jnp.sum(t, axis=1, keepdims=True)
        s2 = jnp.sum(t * t, axis=1, keepdims=True)
        xn = x - (x * s1 - (_N - _K)) / (s1 - x * s2)
        return jnp.abs(xn)

    x = jax.lax.fori_loop(0, _ITERS, body,
                          jnp.full((_B, 1), 1e-6, jnp.float32))
    mask = 1.0 - x / (q + x)
    return s * mask


def _dot_nt(a, b):
    return jax.lax.dot_general(a, b, _NT, preferred_element_type=jnp.float32)


def _fwd(x_ref, w1_ref, b1_ref, w2_ref, b2_ref, w3_ref, b3_ref, w4_ref,
         b4_ref, o_ref):
    s = jnp.maximum(_dot_nt(x_ref[...], w1_ref[...]) + b1_ref[...], 0.0)
    for w_ref, b_ref in ((w2_ref, b2_ref), (w3_ref, b3_ref)):
        h = _soft_topk_mul(s)
        s = jnp.maximum(_dot_nt(h, w_ref[...]) + b_ref[...], 0.0)
    h = _soft_topk_mul(s)
    o_ref[...] = _dot_nt(h, w4_ref[...]) + b4_ref[...]


@jax.jit
def kernel(x, W1, b1, W2, b2, W3, b3, W4, b4):
    return pl.pallas_call(
        _fwd,
        out_shape=jax.ShapeDtypeStruct((_B, W4.shape[0]), jnp.float32),
    )(x, W1, b1.reshape(1, -1), W2, b2.reshape(1, -1), W3, b3.reshape(1, -1),
      W4, b4.reshape(1, -1))
